# baseline (speedup 1.0000x reference)
import jax
import jax.numpy as jnp
from jax import lax
from jax.experimental import pallas as pl
from jax.experimental.pallas import tpu as pltpu
from jax.experimental.pallas import tpu_sc as plsc

N_NODES = 10000
D = 128
W = D // 2
E = 320000
NC = 2
NS = 16
NW = NC * NS
EPW = E // NW
CHUNK = 80
NCHUNK = EPW // CHUNK
GROUPS = CHUNK // 16
WUNROLL = 8


def _body(zs_hbm, zd_hbm, sidx_hbm, didx_hbm, out_hbm,
          sidx_v, didx_v, rows_s0, rows_d0, rows_s1, rows_d1, out_v,
          sem_s0, sem_d0, sem_s1, sem_d1):
    wid = lax.axis_index("s") * NC + lax.axis_index("c")
    base = wid * EPW
    lane = lax.iota(jnp.int32, 16)
    group_rows = [lane + (g * 16) for g in range(GROUPS)]

    pltpu.sync_copy(sidx_hbm.at[pl.ds(base, EPW)], sidx_v)
    pltpu.sync_copy(didx_hbm.at[pl.ds(base, EPW)], didx_v)

    bufs = ((rows_s0, rows_d0, sem_s0, sem_d0),
            (rows_s1, rows_d1, sem_s1, sem_d1))

    def start_gather(c, b):
        rs, rd, ss, sd = bufs[b]
        pltpu.async_copy(zs_hbm.at[sidx_v.at[pl.ds(c * CHUNK, CHUNK)]], rs, ss)
        pltpu.async_copy(zd_hbm.at[didx_v.at[pl.ds(c * CHUNK, CHUNK)]], rd, sd)

    def wait_gather(c, b):
        rs, rd, ss, sd = bufs[b]
        pltpu.make_async_copy(
            zs_hbm.at[sidx_v.at[pl.ds(c * CHUNK, CHUNK)]], rs, ss).wait()
        pltpu.make_async_copy(
            zd_hbm.at[didx_v.at[pl.ds(c * CHUNK, CHUNK)]], rd, sd).wait()

    def compute(c, b):
        rs, rd, _, _ = bufs[b]
        obase = c * CHUNK

        def w_body(k, accs):
            accs = list(accs)
            for u in range(WUNROLL):
                col = jnp.full((16,), k * WUNROLL + u, jnp.int32)
                for g in range(GROUPS):
                    vs = plsc.bitcast(
                        plsc.load_gather(rs, [group_rows[g], col]),
                        jnp.bfloat16)
                    vd = plsc.bitcast(
                        plsc.load_gather(rd, [group_rows[g], col]),
                        jnp.bfloat16)
                    p_lo, p_hi = plsc.unpack(
                        vs * vd, format=plsc.PackFormat.INTERLEAVED)
                    accs[2 * g] = accs[2 * g] + p_lo
                    accs[2 * g + 1] = accs[2 * g + 1] + p_hi
            return tuple(accs)

        zero = jnp.zeros((16,), jnp.float32)
        accs = lax.fori_loop(0, W // WUNROLL, w_body, (zero,) * (2 * GROUPS))
        for g in range(GROUPS):
            out_v[pl.ds(obase + g * 16, 16)] = accs[2 * g] + accs[2 * g + 1]

    start_gather(0, 0)

    def pair_body(k, _):
        c = 2 * k
        start_gather(c + 1, 1)
        wait_gather(c, 0)
        compute(c, 0)
        start_gather(c + 2, 0)
        wait_gather(c + 1, 1)
        compute(c + 1, 1)
        return ()

    lax.fori_loop(0, (NCHUNK - 1) // 2, pair_body, ())
    c_last = NCHUNK - 1
    wait_gather(c_last, 0)
    compute(c_last, 0)

    pltpu.sync_copy(out_v, out_hbm.at[pl.ds(base, EPW)])


@jax.jit
def _run(z_src, z_dst, src_idx, dst_idx):
    mesh = plsc.VectorSubcoreMesh(core_axis_name="c", subcore_axis_name="s")
    f = pl.kernel(
        _body,
        out_type=jax.ShapeDtypeStruct((E,), jnp.float32),
        mesh=mesh,
        compiler_params=pltpu.CompilerParams(
            needs_layout_passes=False, use_tc_tiling_on_sc=False),
        scratch_types=[
            pltpu.VMEM((EPW,), jnp.int32),
            pltpu.VMEM((EPW,), jnp.int32),
            pltpu.VMEM((CHUNK, W), jnp.int32),
            pltpu.VMEM((CHUNK, W), jnp.int32),
            pltpu.VMEM((CHUNK, W), jnp.int32),
            pltpu.VMEM((CHUNK, W), jnp.int32),
            pltpu.VMEM((EPW,), jnp.float32),
            pltpu.SemaphoreType.DMA,
            pltpu.SemaphoreType.DMA,
            pltpu.SemaphoreType.DMA,
            pltpu.SemaphoreType.DMA,
        ],
    )
    return f(z_src, z_dst, src_idx, dst_idx)


def _pack_bf16(z):
    zb = z.astype(jnp.bfloat16).reshape(z.shape[0], z.shape[1] // 2, 2)
    return jax.lax.bitcast_convert_type(zb, jnp.int32)


def kernel(z_src, z_dst, edge_label_index):
    return _run(_pack_bf16(z_src), _pack_bf16(z_dst),
                edge_label_index[0], edge_label_index[1])


# speedup vs baseline: 2.8892x; 2.8892x over previous
import jax
import jax.numpy as jnp
from jax import lax
from jax.experimental import pallas as pl
from jax.experimental.pallas import tpu as pltpu
from jax.experimental.pallas import tpu_sc as plsc

N_NODES = 10000
D = 128
W = D // 2
WP = W + 1
E = 320000
NC = 2
NS = 16
NW = NC * NS
EPW = E // NW
CHUNK = 80
NCHUNK = EPW // CHUNK
GROUPS = CHUNK // 16
WUNROLL = 8


def _body(zs_hbm, zd_hbm, sidx_hbm, didx_hbm, out_hbm,
          sidx_v, didx_v, rows_s0, rows_d0, rows_s1, rows_d1, out_v,
          sem_s0, sem_d0, sem_s1, sem_d1):
    wid = lax.axis_index("s") * NC + lax.axis_index("c")
    base = wid * EPW
    lane = lax.iota(jnp.int32, 16)
    group_rows = [lane + (g * 16) for g in range(GROUPS)]

    pltpu.sync_copy(sidx_hbm.at[pl.ds(base, EPW)], sidx_v)
    pltpu.sync_copy(didx_hbm.at[pl.ds(base, EPW)], didx_v)

    bufs = ((rows_s0, rows_d0, sem_s0, sem_d0),
            (rows_s1, rows_d1, sem_s1, sem_d1))

    def start_gather(c, b):
        rs, rd, ss, sd = bufs[b]
        pltpu.async_copy(zs_hbm.at[sidx_v.at[pl.ds(c * CHUNK, CHUNK)]], rs, ss)
        pltpu.async_copy(zd_hbm.at[didx_v.at[pl.ds(c * CHUNK, CHUNK)]], rd, sd)

    def wait_gather(c, b):
        rs, rd, ss, sd = bufs[b]
        pltpu.make_async_copy(
            zs_hbm.at[sidx_v.at[pl.ds(c * CHUNK, CHUNK)]], rs, ss).wait()
        pltpu.make_async_copy(
            zd_hbm.at[didx_v.at[pl.ds(c * CHUNK, CHUNK)]], rd, sd).wait()

    def compute(c, b):
        rs, rd, _, _ = bufs[b]
        obase = c * CHUNK

        def w_body(k, accs):
            accs = list(accs)
            for u in range(WUNROLL):
                col = jnp.full((16,), k * WUNROLL + u, jnp.int32)
                for g in range(GROUPS):
                    vs = plsc.bitcast(
                        plsc.load_gather(rs, [group_rows[g], col]),
                        jnp.bfloat16)
                    vd = plsc.bitcast(
                        plsc.load_gather(rd, [group_rows[g], col]),
                        jnp.bfloat16)
                    p_lo, p_hi = plsc.unpack(
                        vs * vd, format=plsc.PackFormat.INTERLEAVED)
                    accs[2 * g] = accs[2 * g] + p_lo
                    accs[2 * g + 1] = accs[2 * g + 1] + p_hi
            return tuple(accs)

        zero = jnp.zeros((16,), jnp.float32)
        accs = lax.fori_loop(0, W // WUNROLL, w_body, (zero,) * (2 * GROUPS))
        for g in range(GROUPS):
            out_v[pl.ds(obase + g * 16, 16)] = accs[2 * g] + accs[2 * g + 1]

    start_gather(0, 0)

    def pair_body(k, _):
        c = 2 * k
        start_gather(c + 1, 1)
        wait_gather(c, 0)
        compute(c, 0)
        start_gather(c + 2, 0)
        wait_gather(c + 1, 1)
        compute(c + 1, 1)
        return ()

    lax.fori_loop(0, (NCHUNK - 1) // 2, pair_body, ())
    c_last = NCHUNK - 1
    wait_gather(c_last, 0)
    compute(c_last, 0)

    pltpu.sync_copy(out_v, out_hbm.at[pl.ds(base, EPW)])


@jax.jit
def _run(z_src, z_dst, src_idx, dst_idx):
    mesh = plsc.VectorSubcoreMesh(core_axis_name="c", subcore_axis_name="s")
    f = pl.kernel(
        _body,
        out_type=jax.ShapeDtypeStruct((E,), jnp.float32),
        mesh=mesh,
        compiler_params=pltpu.CompilerParams(
            needs_layout_passes=False, use_tc_tiling_on_sc=False),
        scratch_types=[
            pltpu.VMEM((EPW,), jnp.int32),
            pltpu.VMEM((EPW,), jnp.int32),
            pltpu.VMEM((CHUNK, WP), jnp.int32),
            pltpu.VMEM((CHUNK, WP), jnp.int32),
            pltpu.VMEM((CHUNK, WP), jnp.int32),
            pltpu.VMEM((CHUNK, WP), jnp.int32),
            pltpu.VMEM((EPW,), jnp.float32),
            pltpu.SemaphoreType.DMA,
            pltpu.SemaphoreType.DMA,
            pltpu.SemaphoreType.DMA,
            pltpu.SemaphoreType.DMA,
        ],
    )
    return f(z_src, z_dst, src_idx, dst_idx)


def _pack_bf16(z):
    zb = z.astype(jnp.bfloat16).reshape(z.shape[0], z.shape[1] // 2, 2)
    packed = jax.lax.bitcast_convert_type(zb, jnp.int32)
    return jnp.pad(packed, ((0, 0), (0, WP - W)))


def kernel(z_src, z_dst, edge_label_index):
    return _run(_pack_bf16(z_src), _pack_bf16(z_dst),
                edge_label_index[0], edge_label_index[1])


# speedup vs baseline: 3.5298x; 1.2217x over previous
import jax
import jax.numpy as jnp
from jax import lax
from jax.experimental import pallas as pl
from jax.experimental.pallas import tpu as pltpu
from jax.experimental.pallas import tpu_sc as plsc

N_NODES = 10000
D = 128
W = D // 2
WP = W
E = 320000
NC = 2
NS = 16
NW = NC * NS
EPW = E // NW
CHUNK = 80
NCHUNK = EPW // CHUNK
GROUPS = CHUNK // 16
WUNROLL = 8


def _body(zs_hbm, zd_hbm, sidx_hbm, didx_hbm, out_hbm,
          sidx_v, didx_v, rows_s0, rows_d0, rows_s1, rows_d1, out_v,
          sem_s0, sem_d0, sem_s1, sem_d1):
    wid = lax.axis_index("s") * NC + lax.axis_index("c")
    base = wid * EPW
    lane = lax.iota(jnp.int32, 16)
    group_rows = [lane + (g * 16) for g in range(GROUPS)]

    pltpu.sync_copy(sidx_hbm.at[pl.ds(base, EPW)], sidx_v)
    pltpu.sync_copy(didx_hbm.at[pl.ds(base, EPW)], didx_v)

    bufs = ((rows_s0, rows_d0, sem_s0, sem_d0),
            (rows_s1, rows_d1, sem_s1, sem_d1))

    def start_gather(c, b):
        rs, rd, ss, sd = bufs[b]
        pltpu.async_copy(zs_hbm.at[sidx_v.at[pl.ds(c * CHUNK, CHUNK)]], rs, ss)
        pltpu.async_copy(zd_hbm.at[didx_v.at[pl.ds(c * CHUNK, CHUNK)]], rd, sd)

    def wait_gather(c, b):
        rs, rd, ss, sd = bufs[b]
        pltpu.make_async_copy(
            zs_hbm.at[sidx_v.at[pl.ds(c * CHUNK, CHUNK)]], rs, ss).wait()
        pltpu.make_async_copy(
            zd_hbm.at[didx_v.at[pl.ds(c * CHUNK, CHUNK)]], rd, sd).wait()

    def compute(c, b):
        rs, rd, _, _ = bufs[b]
        obase = c * CHUNK

        def w_body(k, accs):
            accs = list(accs)
            for u in range(WUNROLL):
                col = (lane + (k * WUNROLL + u)) & (W - 1)
                for g in range(GROUPS):
                    vs = plsc.bitcast(
                        plsc.load_gather(rs, [group_rows[g], col]),
                        jnp.bfloat16)
                    vd = plsc.bitcast(
                        plsc.load_gather(rd, [group_rows[g], col]),
                        jnp.bfloat16)
                    p_lo, p_hi = plsc.unpack(
                        vs * vd, format=plsc.PackFormat.INTERLEAVED)
                    accs[2 * g] = accs[2 * g] + p_lo
                    accs[2 * g + 1] = accs[2 * g + 1] + p_hi
            return tuple(accs)

        zero = jnp.zeros((16,), jnp.float32)
        accs = lax.fori_loop(0, W // WUNROLL, w_body, (zero,) * (2 * GROUPS))
        for g in range(GROUPS):
            out_v[pl.ds(obase + g * 16, 16)] = accs[2 * g] + accs[2 * g + 1]

    start_gather(0, 0)

    def pair_body(k, _):
        c = 2 * k
        start_gather(c + 1, 1)
        wait_gather(c, 0)
        compute(c, 0)
        start_gather(c + 2, 0)
        wait_gather(c + 1, 1)
        compute(c + 1, 1)
        return ()

    lax.fori_loop(0, (NCHUNK - 1) // 2, pair_body, ())
    c_last = NCHUNK - 1
    wait_gather(c_last, 0)
    compute(c_last, 0)

    pltpu.sync_copy(out_v, out_hbm.at[pl.ds(base, EPW)])


@jax.jit
def _run(z_src, z_dst, src_idx, dst_idx):
    mesh = plsc.VectorSubcoreMesh(core_axis_name="c", subcore_axis_name="s")
    f = pl.kernel(
        _body,
        out_type=jax.ShapeDtypeStruct((E,), jnp.float32),
        mesh=mesh,
        compiler_params=pltpu.CompilerParams(
            needs_layout_passes=False, use_tc_tiling_on_sc=False),
        scratch_types=[
            pltpu.VMEM((EPW,), jnp.int32),
            pltpu.VMEM((EPW,), jnp.int32),
            pltpu.VMEM((CHUNK, WP), jnp.int32),
            pltpu.VMEM((CHUNK, WP), jnp.int32),
            pltpu.VMEM((CHUNK, WP), jnp.int32),
            pltpu.VMEM((CHUNK, WP), jnp.int32),
            pltpu.VMEM((EPW,), jnp.float32),
            pltpu.SemaphoreType.DMA,
            pltpu.SemaphoreType.DMA,
            pltpu.SemaphoreType.DMA,
            pltpu.SemaphoreType.DMA,
        ],
    )
    return f(z_src, z_dst, src_idx, dst_idx)


def _pack_bf16(z):
    zb = z.astype(jnp.bfloat16).reshape(z.shape[0], z.shape[1] // 2, 2)
    return jax.lax.bitcast_convert_type(zb, jnp.int32)


def kernel(z_src, z_dst, edge_label_index):
    return _run(_pack_bf16(z_src), _pack_bf16(z_dst),
                edge_label_index[0], edge_label_index[1])


# speedup vs baseline: 4.2406x; 1.2014x over previous
import jax
import jax.numpy as jnp
from jax import lax
from jax.experimental import pallas as pl
from jax.experimental.pallas import tpu as pltpu
from jax.experimental.pallas import tpu_sc as plsc

N_NODES = 10000
D = 128
W = D // 2
WP = W
E = 320000
NC = 2
NS = 16
NW = NC * NS
EPW = E // NW
CHUNK = 80
NCHUNK = EPW // CHUNK
GROUPS = CHUNK // 16
WUNROLL = 8


def _body(zs_hbm, zd_hbm, sidx_hbm, didx_hbm, out_hbm,
          sidx_v, didx_v, rows_s0, rows_d0, rows_s1, rows_d1, out_v,
          sem_s0, sem_d0, sem_s1, sem_d1):
    wid = lax.axis_index("s") * NC + lax.axis_index("c")
    base = wid * EPW
    lane = lax.iota(jnp.int32, 16)
    group_rows = [lane + (g * 16) for g in range(GROUPS)]

    pltpu.sync_copy(sidx_hbm.at[pl.ds(base, EPW)], sidx_v)
    pltpu.sync_copy(didx_hbm.at[pl.ds(base, EPW)], didx_v)

    bufs = ((rows_s0, rows_d0, sem_s0, sem_d0),
            (rows_s1, rows_d1, sem_s1, sem_d1))

    def start_gather(c, b):
        rs, rd, ss, sd = bufs[b]
        pltpu.async_copy(zs_hbm.at[sidx_v.at[pl.ds(c * CHUNK, CHUNK)]], rs, ss)
        pltpu.async_copy(zd_hbm.at[didx_v.at[pl.ds(c * CHUNK, CHUNK)]], rd, sd)

    def wait_gather(c, b):
        rs, rd, ss, sd = bufs[b]
        pltpu.make_async_copy(
            zs_hbm.at[sidx_v.at[pl.ds(c * CHUNK, CHUNK)]], rs, ss).wait()
        pltpu.make_async_copy(
            zd_hbm.at[didx_v.at[pl.ds(c * CHUNK, CHUNK)]], rd, sd).wait()

    def compute(c, b):
        rs, rd, _, _ = bufs[b]
        obase = c * CHUNK

        def w_body(k, accs):
            accs = list(accs)
            bacc = [None] * GROUPS
            for u in range(WUNROLL):
                col = (lane + (k * WUNROLL + u)) & (W - 1)
                for g in range(GROUPS):
                    vs = plsc.bitcast(
                        plsc.load_gather(rs, [group_rows[g], col]),
                        jnp.bfloat16)
                    vd = plsc.bitcast(
                        plsc.load_gather(rd, [group_rows[g], col]),
                        jnp.bfloat16)
                    p = vs * vd
                    bacc[g] = p if bacc[g] is None else bacc[g] + p
            for g in range(GROUPS):
                p_lo, p_hi = plsc.unpack(
                    bacc[g], format=plsc.PackFormat.INTERLEAVED)
                accs[2 * g] = accs[2 * g] + p_lo
                accs[2 * g + 1] = accs[2 * g + 1] + p_hi
            return tuple(accs)

        zero = jnp.zeros((16,), jnp.float32)
        accs = lax.fori_loop(0, W // WUNROLL, w_body, (zero,) * (2 * GROUPS))
        for g in range(GROUPS):
            out_v[pl.ds(obase + g * 16, 16)] = accs[2 * g] + accs[2 * g + 1]

    start_gather(0, 0)

    def pair_body(k, _):
        c = 2 * k
        start_gather(c + 1, 1)
        wait_gather(c, 0)
        compute(c, 0)
        start_gather(c + 2, 0)
        wait_gather(c + 1, 1)
        compute(c + 1, 1)
        return ()

    lax.fori_loop(0, (NCHUNK - 1) // 2, pair_body, ())
    c_last = NCHUNK - 1
    wait_gather(c_last, 0)
    compute(c_last, 0)

    pltpu.sync_copy(out_v, out_hbm.at[pl.ds(base, EPW)])


@jax.jit
def _run(z_src, z_dst, src_idx, dst_idx):
    mesh = plsc.VectorSubcoreMesh(core_axis_name="c", subcore_axis_name="s")
    f = pl.kernel(
        _body,
        out_type=jax.ShapeDtypeStruct((E,), jnp.float32),
        mesh=mesh,
        compiler_params=pltpu.CompilerParams(
            needs_layout_passes=False, use_tc_tiling_on_sc=False),
        scratch_types=[
            pltpu.VMEM((EPW,), jnp.int32),
            pltpu.VMEM((EPW,), jnp.int32),
            pltpu.VMEM((CHUNK, WP), jnp.int32),
            pltpu.VMEM((CHUNK, WP), jnp.int32),
            pltpu.VMEM((CHUNK, WP), jnp.int32),
            pltpu.VMEM((CHUNK, WP), jnp.int32),
            pltpu.VMEM((EPW,), jnp.float32),
            pltpu.SemaphoreType.DMA,
            pltpu.SemaphoreType.DMA,
            pltpu.SemaphoreType.DMA,
            pltpu.SemaphoreType.DMA,
        ],
    )
    return f(z_src, z_dst, src_idx, dst_idx)


def _pack_bf16(z):
    zb = z.astype(jnp.bfloat16).reshape(z.shape[0], z.shape[1] // 2, 2)
    return jax.lax.bitcast_convert_type(zb, jnp.int32)


def kernel(z_src, z_dst, edge_label_index):
    return _run(_pack_bf16(z_src), _pack_bf16(z_dst),
                edge_label_index[0], edge_label_index[1])


# speedup vs baseline: 4.3626x; 1.0288x over previous
import jax
import jax.numpy as jnp
from jax import lax
from jax.experimental import pallas as pl
from jax.experimental.pallas import tpu as pltpu
from jax.experimental.pallas import tpu_sc as plsc

N_NODES = 10000
D = 128
W = D // 2
WP = W
E = 320000
NC = 2
NS = 16
NW = NC * NS
EPW = E // NW
CHUNK = 80
NCHUNK = EPW // CHUNK
GROUPS = CHUNK // 16
WUNROLL = 4


def _body(zs_hbm, zd_hbm, sidx_hbm, didx_hbm, out_hbm,
          sidx_v, didx_v, rows_s0, rows_d0, rows_s1, rows_d1, out_v,
          sem_s0, sem_d0, sem_s1, sem_d1):
    wid = lax.axis_index("s") * NC + lax.axis_index("c")
    base = wid * EPW
    lane = lax.iota(jnp.int32, 16)
    group_rows = [lane + (g * 16) for g in range(GROUPS)]

    pltpu.sync_copy(sidx_hbm.at[pl.ds(base, EPW)], sidx_v)
    pltpu.sync_copy(didx_hbm.at[pl.ds(base, EPW)], didx_v)

    bufs = ((rows_s0, rows_d0, sem_s0, sem_d0),
            (rows_s1, rows_d1, sem_s1, sem_d1))

    def start_gather(c, b):
        rs, rd, ss, sd = bufs[b]
        pltpu.async_copy(zs_hbm.at[sidx_v.at[pl.ds(c * CHUNK, CHUNK)]], rs, ss)
        pltpu.async_copy(zd_hbm.at[didx_v.at[pl.ds(c * CHUNK, CHUNK)]], rd, sd)

    def wait_gather(c, b):
        rs, rd, ss, sd = bufs[b]
        pltpu.make_async_copy(
            zs_hbm.at[sidx_v.at[pl.ds(c * CHUNK, CHUNK)]], rs, ss).wait()
        pltpu.make_async_copy(
            zd_hbm.at[didx_v.at[pl.ds(c * CHUNK, CHUNK)]], rd, sd).wait()

    def compute(c, b):
        rs, rd, _, _ = bufs[b]
        obase = c * CHUNK

        def w_body(k, accs):
            accs = list(accs)
            bacc = [None] * GROUPS
            for u in range(WUNROLL):
                col = (lane + (k * WUNROLL + u)) & (W - 1)
                for g in range(GROUPS):
                    vs = plsc.bitcast(
                        plsc.load_gather(rs, [group_rows[g], col]),
                        jnp.bfloat16)
                    vd = plsc.bitcast(
                        plsc.load_gather(rd, [group_rows[g], col]),
                        jnp.bfloat16)
                    p = vs * vd
                    bacc[g] = p if bacc[g] is None else bacc[g] + p
            for g in range(GROUPS):
                p_lo, p_hi = plsc.unpack(
                    bacc[g], format=plsc.PackFormat.INTERLEAVED)
                accs[g] = accs[g] + (p_lo + p_hi)
            return tuple(accs)

        zero = jnp.zeros((16,), jnp.float32)
        accs = lax.fori_loop(0, W // WUNROLL, w_body, (zero,) * GROUPS)
        for g in range(GROUPS):
            out_v[pl.ds(obase + g * 16, 16)] = accs[g]

    start_gather(0, 0)

    def pair_body(k, _):
        c = 2 * k
        start_gather(c + 1, 1)
        wait_gather(c, 0)
        compute(c, 0)
        start_gather(c + 2, 0)
        wait_gather(c + 1, 1)
        compute(c + 1, 1)
        return ()

    lax.fori_loop(0, (NCHUNK - 1) // 2, pair_body, ())
    if NCHUNK % 2 == 1:
        wait_gather(NCHUNK - 1, 0)
        compute(NCHUNK - 1, 0)
    else:
        start_gather(NCHUNK - 1, 1)
        wait_gather(NCHUNK - 2, 0)
        compute(NCHUNK - 2, 0)
        wait_gather(NCHUNK - 1, 1)
        compute(NCHUNK - 1, 1)

    pltpu.sync_copy(out_v, out_hbm.at[pl.ds(base, EPW)])


@jax.jit
def _run(z_src, z_dst, src_idx, dst_idx):
    mesh = plsc.VectorSubcoreMesh(core_axis_name="c", subcore_axis_name="s")
    f = pl.kernel(
        _body,
        out_type=jax.ShapeDtypeStruct((E,), jnp.float32),
        mesh=mesh,
        compiler_params=pltpu.CompilerParams(
            needs_layout_passes=False, use_tc_tiling_on_sc=False),
        scratch_types=[
            pltpu.VMEM((EPW,), jnp.int32),
            pltpu.VMEM((EPW,), jnp.int32),
            pltpu.VMEM((CHUNK, WP), jnp.int32),
            pltpu.VMEM((CHUNK, WP), jnp.int32),
            pltpu.VMEM((CHUNK, WP), jnp.int32),
            pltpu.VMEM((CHUNK, WP), jnp.int32),
            pltpu.VMEM((EPW,), jnp.float32),
            pltpu.SemaphoreType.DMA,
            pltpu.SemaphoreType.DMA,
            pltpu.SemaphoreType.DMA,
            pltpu.SemaphoreType.DMA,
        ],
    )
    return f(z_src, z_dst, src_idx, dst_idx)


def _pack_bf16(z):
    zb = z.astype(jnp.bfloat16).reshape(z.shape[0], z.shape[1] // 2, 2)
    return jax.lax.bitcast_convert_type(zb, jnp.int32)


def kernel(z_src, z_dst, edge_label_index):
    return _run(_pack_bf16(z_src), _pack_bf16(z_dst),
                edge_label_index[0], edge_label_index[1])
